# baseline (speedup 1.0000x reference)
import functools

import jax
import jax.numpy as jnp
from jax.experimental import pallas as pl
from jax.experimental.pallas import tpu as pltpu


def _cc_kernel(left_ref, right_ref, win_ref, wck_ref, br_ref, out_ref,
               ys_ref, xs_ref, *, L, NB, T, NSTEPS):
    f32 = jnp.float32
    bf16 = jnp.bfloat16
    S = L + 1

    y3 = ys_ref[...]
    y0 = y3[:, 0:L, :].reshape(NB * L, T)
    y1 = y3[:, L:2 * L, :].reshape(NB * L, T)
    y2 = y3[:, 2 * L:, :].reshape(NB * L, T)
    t = jax.lax.broadcasted_iota(jnp.int32, (1, T), 1)
    not_first = (t != 0).astype(f32)
    not_last = (t != T - 1).astype(f32)
    out_ref[...] = (y1
                    + not_first * pltpu.roll(y0, 1, axis=1)
                    + not_last * pltpu.roll(y2, T - 1, axis=1))

    ps = jnp.dot(win_ref[...], xs_ref[...],
                 preferred_element_type=f32)
    ps3 = ps.reshape(NB, S * L, T)
    sf = ps3[:, 0:L, :] + br_ref[...]
    sfb = sf.astype(bf16)
    sfx = jnp.broadcast_to(sfb[:, None, :, :], (NB, L, L, T))
    prods = ps3[:, L:, :].astype(bf16) * sfx.reshape(NB, L * L, T)
    prodall = jnp.concatenate([sfb, prods],
                              axis=1).reshape(NB * S * L, T)
    yall = jnp.dot(wck_ref[...], prodall,
                   preferred_element_type=f32)
    ys_ref[...] = yall.reshape(NB, 3 * L, T)

    xs_ref[...] = jnp.concatenate(
        [left_ref[...].astype(bf16).reshape(NB, L, T),
         right_ref[...].astype(bf16).reshape(NB, L, T)],
        axis=1).reshape(NB * 2 * L, T)


def _pick_nb(B, L, T):
    best = 1
    for nb in range(1, B + 1):
        if B % nb:
            continue
        rows = nb * L
        if rows > 128 or rows % 8:
            continue
        if nb * L * T * 4 > 2 * 1024 * 1024:
            continue
        best = nb
    return best


def kernel(left, right, wl, bl, wr, br, wconv):
    B, L, T = left.shape
    f32 = jnp.float32
    bf16 = jnp.bfloat16
    NB = _pick_nb(B, L, T)
    R = NB * L
    S = L + 1

    wl_f = wl.astype(f32)
    bl_f = bl.astype(f32)
    wc_f = wconv.astype(f32)
    eye = jnp.eye(NB, dtype=bf16)

    q_sf = jnp.pad(wr.astype(f32)[::-1, :], ((0, 0), (L, 0)))
    q_p = jnp.pad(jnp.broadcast_to(wl_f[:, None, :], (L, L, L)),
                  ((0, 0), (0, 0), (0, L))).reshape(L * L, 2 * L)
    QIN = jnp.concatenate([q_sf, q_p], axis=0)
    WIN = jnp.kron(eye, QIN.astype(bf16))

    idx = jnp.arange(L)[:, None] + jnp.arange(L)[None, :]
    win4 = wc_f[:, idx, :]
    ck = jnp.einsum('j,cjmk->kcm', bl_f, win4)[:, :, None, :]
    qc = jnp.concatenate([ck, win4.transpose(3, 0, 1, 2)],
                         axis=2)
    QC = qc.reshape(3 * L, S * L)
    WCK = jnp.kron(eye, QC.astype(bf16))

    left2 = left.astype(f32).reshape(B * L, T)
    right2 = right.astype(f32).reshape(B * L, T)
    br_in = br.astype(f32)[::-1].reshape(L, 1)

    NSTEPS = B // NB
    io_in = pl.BlockSpec((R, T), lambda i: (jnp.minimum(i, NSTEPS - 1), 0))
    io_out = pl.BlockSpec((R, T), lambda i: (jnp.maximum(i - 2, 0), 0))
    cst = lambda shape: pl.BlockSpec(shape, lambda i: (0, 0))

    out2 = pl.pallas_call(
        functools.partial(_cc_kernel, L=L, NB=NB, T=T, NSTEPS=NSTEPS),
        out_shape=jax.ShapeDtypeStruct((B * L, T), f32),
        grid=(NSTEPS + 2,),
        in_specs=[io_in, io_in,
                  cst((S * R, 2 * R)), cst((3 * R, S * R)), cst((L, 1))],
        out_specs=io_out,
        scratch_shapes=[pltpu.VMEM((NB, 3 * L, T), f32),
                        pltpu.VMEM((NB * 2 * L, T), jnp.bfloat16)],
        compiler_params=pltpu.CompilerParams(
            dimension_semantics=("arbitrary",),
            vmem_limit_bytes=64 * 1024 * 1024),
    )(left2, right2, WIN, WCK, br_in)
    return out2.reshape(B, L, T)


# speedup vs baseline: 1.0166x; 1.0166x over previous
import functools

import jax
import jax.numpy as jnp
from jax.experimental import pallas as pl
from jax.experimental.pallas import tpu as pltpu


def _cc_kernel(left_ref, right_ref, win_ref, wck_ref, br_ref, out_ref,
               ys_ref, *, L, NB, T, NSTEPS):
    f32 = jnp.float32
    bf16 = jnp.bfloat16
    S = L + 1

    y3 = ys_ref[...]
    y0 = y3[:, 0:L, :].reshape(NB * L, T)
    y1 = y3[:, L:2 * L, :].reshape(NB * L, T)
    y2 = y3[:, 2 * L:, :].reshape(NB * L, T)
    t = jax.lax.broadcasted_iota(jnp.int32, (1, T), 1)
    not_first = (t != 0).astype(f32)
    not_last = (t != T - 1).astype(f32)
    out_ref[...] = (y1
                    + not_first * pltpu.roll(y0, 1, axis=1)
                    + not_last * pltpu.roll(y2, T - 1, axis=1))

    x = jnp.concatenate(
        [left_ref[...].astype(bf16).reshape(NB, L, T),
         right_ref[...].astype(bf16).reshape(NB, L, T)],
        axis=1).reshape(NB * 2 * L, T)
    ps = jnp.dot(win_ref[...], x,
                 preferred_element_type=f32)
    ps3 = ps.reshape(NB, S * L, T)
    sf = ps3[:, 0:L, :] + br_ref[...]
    sfb = sf.astype(bf16)
    sfx = jnp.broadcast_to(sfb[:, None, :, :], (NB, L, L, T))
    prods = ps3[:, L:, :].astype(bf16) * sfx.reshape(NB, L * L, T)
    prodall = jnp.concatenate([sfb, prods],
                              axis=1).reshape(NB * S * L, T)
    yall = jnp.dot(wck_ref[...], prodall,
                   preferred_element_type=f32)
    ys_ref[...] = yall.reshape(NB, 3 * L, T)


def _pick_nb(B, L, T):
    best = 1
    for nb in range(1, B + 1):
        if B % nb:
            continue
        rows = nb * L
        if rows > 128 or rows % 8:
            continue
        if nb * L * T * 4 > 2 * 1024 * 1024:
            continue
        best = nb
    return best


def kernel(left, right, wl, bl, wr, br, wconv):
    B, L, T = left.shape
    f32 = jnp.float32
    bf16 = jnp.bfloat16
    NB = _pick_nb(B, L, T)
    R = NB * L
    S = L + 1

    wl_f = wl.astype(f32)
    bl_f = bl.astype(f32)
    wc_f = wconv.astype(f32)
    eye = jnp.eye(NB, dtype=bf16)

    q_sf = jnp.pad(wr.astype(f32)[::-1, :], ((0, 0), (L, 0)))
    q_p = jnp.pad(jnp.broadcast_to(wl_f[:, None, :], (L, L, L)),
                  ((0, 0), (0, 0), (0, L))).reshape(L * L, 2 * L)
    QIN = jnp.concatenate([q_sf, q_p], axis=0)
    WIN = jnp.kron(eye, QIN.astype(bf16))

    idx = jnp.arange(L)[:, None] + jnp.arange(L)[None, :]
    win4 = wc_f[:, idx, :]
    ck = jnp.einsum('j,cjmk->kcm', bl_f, win4)[:, :, None, :]
    qc = jnp.concatenate([ck, win4.transpose(3, 0, 1, 2)],
                         axis=2)
    QC = qc.reshape(3 * L, S * L)
    WCK = jnp.kron(eye, QC.astype(bf16))

    left2 = left.astype(f32).reshape(B * L, T)
    right2 = right.astype(f32).reshape(B * L, T)
    br_in = br.astype(f32)[::-1].reshape(L, 1)

    NSTEPS = B // NB
    io_in = pl.BlockSpec((R, T), lambda i: (jnp.minimum(i, NSTEPS - 1), 0))
    io_out = pl.BlockSpec((R, T), lambda i: (jnp.maximum(i - 1, 0), 0))
    cst = lambda shape: pl.BlockSpec(shape, lambda i: (0, 0))

    out2 = pl.pallas_call(
        functools.partial(_cc_kernel, L=L, NB=NB, T=T, NSTEPS=NSTEPS),
        out_shape=jax.ShapeDtypeStruct((B * L, T), f32),
        grid=(NSTEPS + 1,),
        in_specs=[io_in, io_in,
                  cst((S * R, 2 * R)), cst((3 * R, S * R)), cst((L, 1))],
        out_specs=io_out,
        scratch_shapes=[pltpu.VMEM((NB, 3 * L, T), f32)],
        compiler_params=pltpu.CompilerParams(
            dimension_semantics=("arbitrary",),
            vmem_limit_bytes=64 * 1024 * 1024),
    )(left2, right2, WIN, WCK, br_in)
    return out2.reshape(B, L, T)
